# Initial kernel scaffold; baseline (speedup 1.0000x reference)
import jax
import jax.numpy as jnp
from jax import lax
from jax.experimental import pallas as pl
from jax.experimental.pallas import tpu as pltpu
from jax.experimental.pallas import tpu_sc as plsc

N = 10000
E = 320000
D = 128
NC = 2
NS = 16
NW = NC * NS
L = 16
CHUNK = 128
EPT = -(-E // (NW * CHUNK)) * CHUNK
E_PAD = EPT * NW
NACC = 10240
DEGW = 16
NAB = N + 16
INIT_ROWS = NACC // NS
OUT_ROWS = N // NS
BLK = 2000


def _mesh():
  return plsc.VectorSubcoreMesh(
      core_axis_name="c", subcore_axis_name="s", num_cores=NC, num_subcores=NS)


def _agg_deg(table, srcb, dstb, z128, z16, ones):
  def body(table, srcb, dstb, z128, z16, ones, out, degout,
           acc, dacc, idx_s, idx_d, rows, ones_v, sem):
    c = lax.axis_index("c")
    s = lax.axis_index("s")
    wid = s * NC + c
    r0 = s * INIT_ROWS
    pltpu.sync_copy(z128.at[pl.ds(r0, INIT_ROWS)], acc.at[pl.ds(r0, INIT_ROWS)])
    pltpu.sync_copy(z16.at[pl.ds(r0, INIT_ROWS)], dacc.at[pl.ds(r0, INIT_ROWS)])
    pltpu.sync_copy(ones, ones_v)
    plsc.subcore_barrier()
    base = wid * EPT

    def step(i, carry):
      off = base + i * CHUNK
      pltpu.sync_copy(srcb.at[pl.ds(off, CHUNK)], idx_s)
      pltpu.sync_copy(dstb.at[pl.ds(off, CHUNK)], idx_d)
      pltpu.async_copy(table.at[idx_s], rows, sem).wait()
      pltpu.sync_copy(rows, acc.at[idx_d], add=True)
      pltpu.sync_copy(ones_v, dacc.at[idx_d], add=True)
      return carry

    lax.fori_loop(0, EPT // CHUNK, step, 0)
    plsc.subcore_barrier()
    o0 = s * OUT_ROWS
    pltpu.sync_copy(acc.at[pl.ds(o0, OUT_ROWS)], out.at[c, pl.ds(o0, OUT_ROWS)])
    pltpu.sync_copy(dacc.at[pl.ds(o0, OUT_ROWS)], degout.at[c, pl.ds(o0, OUT_ROWS)])

  return pl.kernel(
      body,
      out_type=(jax.ShapeDtypeStruct((NC, N, D), jnp.float32),
                jax.ShapeDtypeStruct((NC, N, DEGW), jnp.float32)),
      mesh=_mesh(),
      scratch_types=[
          pltpu.VMEM_SHARED((NACC, D), jnp.float32),
          pltpu.VMEM_SHARED((NACC, DEGW), jnp.float32),
          pltpu.VMEM((CHUNK,), jnp.int32),
          pltpu.VMEM((CHUNK,), jnp.int32),
          pltpu.VMEM((CHUNK, D), jnp.float32),
          pltpu.VMEM((CHUNK, DEGW), jnp.float32),
          pltpu.SemaphoreType.DMA,
      ],
  )(table, srcb, dstb, z128, z16, ones)


def _agg(table, srcb, dstb, z128):
  def body(table, srcb, dstb, z128, out, acc, idx_s, idx_d, rows, sem):
    c = lax.axis_index("c")
    s = lax.axis_index("s")
    wid = s * NC + c
    r0 = s * INIT_ROWS
    pltpu.sync_copy(z128.at[pl.ds(r0, INIT_ROWS)], acc.at[pl.ds(r0, INIT_ROWS)])
    plsc.subcore_barrier()
    base = wid * EPT

    def step(i, carry):
      off = base + i * CHUNK
      pltpu.sync_copy(srcb.at[pl.ds(off, CHUNK)], idx_s)
      pltpu.sync_copy(dstb.at[pl.ds(off, CHUNK)], idx_d)
      pltpu.async_copy(table.at[idx_s], rows, sem).wait()
      pltpu.sync_copy(rows, acc.at[idx_d], add=True)
      return carry

    lax.fori_loop(0, EPT // CHUNK, step, 0)
    plsc.subcore_barrier()
    o0 = s * OUT_ROWS
    pltpu.sync_copy(acc.at[pl.ds(o0, OUT_ROWS)], out.at[c, pl.ds(o0, OUT_ROWS)])

  return pl.kernel(
      body,
      out_type=jax.ShapeDtypeStruct((NC, N, D), jnp.float32),
      mesh=_mesh(),
      scratch_types=[
          pltpu.VMEM_SHARED((NACC, D), jnp.float32),
          pltpu.VMEM((CHUNK,), jnp.int32),
          pltpu.VMEM((CHUNK,), jnp.int32),
          pltpu.VMEM((CHUNK, D), jnp.float32),
          pltpu.SemaphoreType.DMA,
      ],
  )(table, srcb, dstb, z128)


def _tc_layer1(x, na, nb, da, db, Ws, Wn, bs, bn):
  def body(x_r, na_r, nb_r, da_r, db_r, ws_r, wn_r, b_r, out_r):
    deg = da_r[:, 0:1] + db_r[:, 0:1]
    inv = 1.0 / jnp.maximum(deg, 1.0)
    neigh = (na_r[...] + nb_r[...]) * inv
    h = (jnp.dot(x_r[...], ws_r[...], preferred_element_type=jnp.float32)
         + jnp.dot(neigh, wn_r[...], preferred_element_type=jnp.float32)
         + b_r[...])
    out_r[...] = jnp.maximum(h, 0.0)

  row = lambda i: (i, 0)
  fix = lambda i: (0, 0)
  return pl.pallas_call(
      body,
      grid=(N // BLK,),
      in_specs=[
          pl.BlockSpec((BLK, D), row),
          pl.BlockSpec((BLK, D), row),
          pl.BlockSpec((BLK, D), row),
          pl.BlockSpec((BLK, DEGW), row),
          pl.BlockSpec((BLK, DEGW), row),
          pl.BlockSpec((D, D), fix),
          pl.BlockSpec((D, D), fix),
          pl.BlockSpec((1, D), fix),
      ],
      out_specs=pl.BlockSpec((BLK, D), row),
      out_shape=jax.ShapeDtypeStruct((N, D), jnp.float32),
  )(x, na, nb, da, db, Ws, Wn, (bs + bn).reshape(1, D))


def _tc_layer2(h1, na, nb, da, db, Ws, Wn, bs, bn, Wpc, bvec):
  def body(h_r, na_r, nb_r, da_r, db_r, ws_r, wn_r, b_r, wp_r, bv_r, out_r):
    deg = da_r[:, 0:1] + db_r[:, 0:1]
    inv = 1.0 / jnp.maximum(deg, 1.0)
    neigh = (na_r[...] + nb_r[...]) * inv
    h2 = (jnp.dot(h_r[...], ws_r[...], preferred_element_type=jnp.float32)
          + jnp.dot(neigh, wn_r[...], preferred_element_type=jnp.float32)
          + b_r[...])
    out_r[...] = jnp.dot(h2, wp_r[...], preferred_element_type=jnp.float32) + bv_r[...]

  row = lambda i: (i, 0)
  fix = lambda i: (0, 0)
  return pl.pallas_call(
      body,
      grid=(N // BLK,),
      in_specs=[
          pl.BlockSpec((BLK, D), row),
          pl.BlockSpec((BLK, D), row),
          pl.BlockSpec((BLK, D), row),
          pl.BlockSpec((BLK, DEGW), row),
          pl.BlockSpec((BLK, DEGW), row),
          pl.BlockSpec((D, D), fix),
          pl.BlockSpec((D, D), fix),
          pl.BlockSpec((1, D), fix),
          pl.BlockSpec((D, 2), fix),
          pl.BlockSpec((1, 2), fix),
      ],
      out_specs=pl.BlockSpec((BLK, 2), row),
      out_shape=jax.ShapeDtypeStruct((N, 2), jnp.float32),
  )(h1, na, nb, da, db, Ws, Wn, (bs + bn).reshape(1, D), Wpc, bvec)


def _edge_score(ab, srcb, dstb):
  def body(ab, srcb, dstb, out, ab_v, src_v, dst_v, out_v):
    c = lax.axis_index("c")
    s = lax.axis_index("s")
    wid = s * NC + c
    base = wid * EPT
    pltpu.sync_copy(ab, ab_v)
    pltpu.sync_copy(srcb.at[pl.ds(base, EPT)], src_v)
    pltpu.sync_copy(dstb.at[pl.ds(base, EPT)], dst_v)
    col0 = jnp.zeros((L,), jnp.int32)
    col1 = jnp.ones((L,), jnp.int32)

    def step(i, carry):
      si = src_v[pl.ds(i * L, L)]
      di = dst_v[pl.ds(i * L, L)]
      av = plsc.load_gather(ab_v, [si, col0])
      bv = plsc.load_gather(ab_v, [di, col1])
      out_v[pl.ds(i * L, L)] = av + bv
      return carry

    lax.fori_loop(0, EPT // L, step, 0)
    pltpu.sync_copy(out_v, out.at[pl.ds(base, EPT)])

  return pl.kernel(
      body,
      out_type=jax.ShapeDtypeStruct((E_PAD,), jnp.float32),
      mesh=_mesh(),
      scratch_types=[
          pltpu.VMEM((NAB, 2), jnp.float32),
          pltpu.VMEM((EPT,), jnp.int32),
          pltpu.VMEM((EPT,), jnp.int32),
          pltpu.VMEM((EPT,), jnp.float32),
      ],
  )(ab, srcb, dstb)


def kernel(x, edge_index, W1s, b1s, W1n, b1n, W2s, b2s, W2n, b2n, Wp, bp):
  src = edge_index[0]
  dst = edge_index[1]
  pad = E_PAD - E
  srcb = jnp.concatenate([src, jnp.zeros((pad,), jnp.int32)])
  dstb = jnp.concatenate([dst, jnp.full((pad,), N, jnp.int32)])
  z128 = jnp.zeros((NACC, D), jnp.float32)
  z16 = jnp.zeros((NACC, DEGW), jnp.float32)
  ones = jnp.ones((CHUNK, DEGW), jnp.float32)

  n1p, degp = _agg_deg(x, srcb, dstb, z128, z16, ones)
  h1 = _tc_layer1(x, n1p[0], n1p[1], degp[0], degp[1], W1s, W1n, b1s, b1n)
  n2p = _agg(h1, srcb, dstb, z128)
  Wpc = jnp.concatenate([Wp[:D], Wp[D:]], axis=1)
  bvec = jnp.concatenate([jnp.zeros((1,), jnp.float32), bp]).reshape(1, 2)
  ab = _tc_layer2(h1, n2p[0], n2p[1], degp[0], degp[1], W2s, W2n, b2s, b2n,
                  Wpc, bvec)
  ab_pad = jnp.concatenate([ab, jnp.zeros((NAB - N, 2), jnp.float32)])
  score = _edge_score(ab_pad, srcb, dstb)
  return score[:E].reshape(E, 1)


# speedup vs baseline: 5.6747x; 5.6747x over previous
import jax
import jax.numpy as jnp
from jax import lax
from jax.experimental import pallas as pl
from jax.experimental.pallas import tpu as pltpu
from jax.experimental.pallas import tpu_sc as plsc

N = 10000
E = 320000
D = 128
NC = 2
NS = 16
NW = NC * NS
L = 16
CHUNK = 128
EPT = -(-E // (NW * CHUNK)) * CHUNK
E_PAD = EPT * NW
N_PAD = 10112
NACC = 10240
INIT_ROWS = NACC // NS
OUT_ROWS = N_PAD // NS
BLK = 1264


def _mesh():
  return plsc.VectorSubcoreMesh(
      core_axis_name="c", subcore_axis_name="s", num_cores=NC, num_subcores=NS)


def _deg(dstb, z128, ones128):
  def body(dstb, z128, ones128, out, acc, idx_d, ones_v, sem):
    c = lax.axis_index("c")
    s = lax.axis_index("s")
    wid = s * NC + c
    r0 = s * INIT_ROWS
    pltpu.sync_copy(z128.at[pl.ds(r0, INIT_ROWS)], acc.at[pl.ds(r0, INIT_ROWS)])
    pltpu.sync_copy(ones128, ones_v)
    plsc.subcore_barrier()
    base = wid * EPT

    def step(i, carry):
      off = base + i * CHUNK
      pltpu.sync_copy(dstb.at[pl.ds(off, CHUNK)], idx_d)
      pltpu.sync_copy(ones_v, acc.at[idx_d], add=True)
      return carry

    lax.fori_loop(0, EPT // CHUNK, step, 0)
    plsc.subcore_barrier()
    o0 = s * OUT_ROWS
    pltpu.sync_copy(acc.at[pl.ds(o0, OUT_ROWS)], out.at[c, pl.ds(o0, OUT_ROWS)])

  return pl.kernel(
      body,
      out_type=jax.ShapeDtypeStruct((NC, N_PAD, D), jnp.float32),
      mesh=_mesh(),
      scratch_types=[
          pltpu.VMEM_SHARED((NACC, D), jnp.float32),
          pltpu.VMEM((CHUNK,), jnp.int32),
          pltpu.VMEM((CHUNK, D), jnp.float32),
          pltpu.SemaphoreType.DMA,
      ],
  )(dstb, z128, ones128)


def _agg(table, srcb, dstb, z128):
  def body(table, srcb, dstb, z128, out, acc, idx_s, idx_d, rows, sem):
    c = lax.axis_index("c")
    s = lax.axis_index("s")
    wid = s * NC + c
    r0 = s * INIT_ROWS
    pltpu.sync_copy(z128.at[pl.ds(r0, INIT_ROWS)], acc.at[pl.ds(r0, INIT_ROWS)])
    plsc.subcore_barrier()
    base = wid * EPT

    def step(i, carry):
      off = base + i * CHUNK
      pltpu.sync_copy(srcb.at[pl.ds(off, CHUNK)], idx_s)
      pltpu.sync_copy(dstb.at[pl.ds(off, CHUNK)], idx_d)
      pltpu.async_copy(table.at[idx_s], rows, sem).wait()
      pltpu.sync_copy(rows, acc.at[idx_d], add=True)
      return carry

    lax.fori_loop(0, EPT // CHUNK, step, 0)
    plsc.subcore_barrier()
    o0 = s * OUT_ROWS
    pltpu.sync_copy(acc.at[pl.ds(o0, OUT_ROWS)], out.at[c, pl.ds(o0, OUT_ROWS)])

  return pl.kernel(
      body,
      out_type=jax.ShapeDtypeStruct((NC, N_PAD, D), jnp.float32),
      mesh=_mesh(),
      scratch_types=[
          pltpu.VMEM_SHARED((NACC, D), jnp.float32),
          pltpu.VMEM((CHUNK,), jnp.int32),
          pltpu.VMEM((CHUNK,), jnp.int32),
          pltpu.VMEM((CHUNK, D), jnp.float32),
          pltpu.SemaphoreType.DMA,
      ],
  )(table, srcb, dstb, z128)


def _tc_layer1(x, na, nb, da, db, Ws, Wn, bs, bn):
  def body(x_r, na_r, nb_r, da_r, db_r, ws_r, wn_r, b_r, out_r):
    deg = da_r[:, 0:1] + db_r[:, 0:1]
    inv = 1.0 / jnp.maximum(deg, 1.0)
    neigh = (na_r[...] + nb_r[...]) * inv
    h = (jnp.dot(x_r[...], ws_r[...], preferred_element_type=jnp.float32)
         + jnp.dot(neigh, wn_r[...], preferred_element_type=jnp.float32)
         + b_r[...])
    out_r[...] = jnp.maximum(h, 0.0)

  row = lambda i: (i, 0)
  fix = lambda i: (0, 0)
  return pl.pallas_call(
      body,
      grid=(N_PAD // BLK,),
      in_specs=[
          pl.BlockSpec((BLK, D), row),
          pl.BlockSpec((BLK, D), row),
          pl.BlockSpec((BLK, D), row),
          pl.BlockSpec((BLK, D), row),
          pl.BlockSpec((BLK, D), row),
          pl.BlockSpec((D, D), fix),
          pl.BlockSpec((D, D), fix),
          pl.BlockSpec((1, D), fix),
      ],
      out_specs=pl.BlockSpec((BLK, D), row),
      out_shape=jax.ShapeDtypeStruct((N_PAD, D), jnp.float32),
  )(x, na, nb, da, db, Ws, Wn, (bs + bn).reshape(1, D))


def _tc_layer2(h1, na, nb, da, db, Ws, Wn, bs, bn, Wpc, bvec):
  def body(h_r, na_r, nb_r, da_r, db_r, ws_r, wn_r, b_r, wp_r, bv_r, out_r):
    deg = da_r[:, 0:1] + db_r[:, 0:1]
    inv = 1.0 / jnp.maximum(deg, 1.0)
    neigh = (na_r[...] + nb_r[...]) * inv
    h2 = (jnp.dot(h_r[...], ws_r[...], preferred_element_type=jnp.float32)
          + jnp.dot(neigh, wn_r[...], preferred_element_type=jnp.float32)
          + b_r[...])
    out_r[...] = jnp.dot(h2, wp_r[...], preferred_element_type=jnp.float32) + bv_r[...]

  row = lambda i: (i, 0)
  fix = lambda i: (0, 0)
  return pl.pallas_call(
      body,
      grid=(N_PAD // BLK,),
      in_specs=[
          pl.BlockSpec((BLK, D), row),
          pl.BlockSpec((BLK, D), row),
          pl.BlockSpec((BLK, D), row),
          pl.BlockSpec((BLK, D), row),
          pl.BlockSpec((BLK, D), row),
          pl.BlockSpec((D, D), fix),
          pl.BlockSpec((D, D), fix),
          pl.BlockSpec((1, D), fix),
          pl.BlockSpec((D, 2), fix),
          pl.BlockSpec((1, 2), fix),
      ],
      out_specs=pl.BlockSpec((BLK, 2), row),
      out_shape=jax.ShapeDtypeStruct((N_PAD, 2), jnp.float32),
  )(h1, na, nb, da, db, Ws, Wn, (bs + bn).reshape(1, D), Wpc, bvec)


def _edge_score(ab, srcb, dstb):
  def body(ab, srcb, dstb, out, ab_v, src_v, dst_v, out_v):
    c = lax.axis_index("c")
    s = lax.axis_index("s")
    wid = s * NC + c
    base = wid * EPT
    pltpu.sync_copy(ab, ab_v)
    pltpu.sync_copy(srcb.at[pl.ds(base, EPT)], src_v)
    pltpu.sync_copy(dstb.at[pl.ds(base, EPT)], dst_v)

    def step(i, carry):
      si = src_v[pl.ds(i * L, L)]
      di = dst_v[pl.ds(i * L, L)]
      av = plsc.load_gather(ab_v, [si * 2])
      bv = plsc.load_gather(ab_v, [di * 2 + 1])
      out_v[pl.ds(i * L, L)] = av + bv
      return carry

    lax.fori_loop(0, EPT // L, step, 0)
    pltpu.sync_copy(out_v, out.at[pl.ds(base, EPT)])

  return pl.kernel(
      body,
      out_type=jax.ShapeDtypeStruct((E_PAD,), jnp.float32),
      mesh=_mesh(),
      compiler_params=pltpu.CompilerParams(needs_layout_passes=False),
      scratch_types=[
          pltpu.VMEM((N_PAD * 2,), jnp.float32),
          pltpu.VMEM((EPT,), jnp.int32),
          pltpu.VMEM((EPT,), jnp.int32),
          pltpu.VMEM((EPT,), jnp.float32),
      ],
  )(ab, srcb, dstb)


def kernel(x, edge_index, W1s, b1s, W1n, b1n, W2s, b2s, W2n, b2n, Wp, bp):
  src = edge_index[0]
  dst = edge_index[1]
  pad = E_PAD - E
  srcb = jnp.concatenate([src, jnp.zeros((pad,), jnp.int32)])
  dstb = jnp.concatenate([dst, jnp.full((pad,), N, jnp.int32)])
  z128 = jnp.zeros((NACC, D), jnp.float32)
  ones128 = jnp.ones((CHUNK, D), jnp.float32)

  x_pad = jnp.concatenate([x, jnp.zeros((N_PAD - N, D), jnp.float32)])
  degp = _deg(dstb, z128, ones128)
  n1p = _agg(x, srcb, dstb, z128)
  h1 = _tc_layer1(x_pad, n1p[0], n1p[1], degp[0], degp[1], W1s, W1n, b1s, b1n)
  n2p = _agg(h1, srcb, dstb, z128)
  Wpc = jnp.concatenate([Wp[:D], Wp[D:]], axis=1)
  bvec = jnp.concatenate([jnp.zeros((1,), jnp.float32), bp]).reshape(1, 2)
  ab = _tc_layer2(h1, n2p[0], n2p[1], degp[0], degp[1], W2s, W2n, b2s, b2n,
                  Wpc, bvec)
  score = _edge_score(ab.reshape(N_PAD * 2), srcb, dstb)
  return score[:E].reshape(E, 1)
